# baseline (speedup 1.0000x reference)
import functools

import jax
import jax.numpy as jnp
import numpy as np
from jax import lax
from jax.experimental import pallas as pl
from jax.experimental.pallas import tpu as pltpu
from jax.experimental.pallas import tpu_sc as plsc

N_NODES = 10000
N_EDGES = 160000
C_N = 256
C_Z = 128
BIAS = 128
HALF = BIAS // 2
HID = 384

PARTS = (9600, 22400, 41600, 43200, 43200)
TE = 1600
CH = 128

_PERM = np.concatenate([
    np.arange(0, 128),
    np.arange(128, 192),
    np.arange(256, 320),
    np.arange(192, 256),
    np.arange(320, 384),
])


def _rtne16(u):
    return (u + 0x7FFF + ((u >> 16) & 1)) >> 16


def _embed_body(s_ref, w0_ref, b0_ref, o_ref):
    o = (jnp.dot(s_ref[...], w0_ref[...], preferred_element_type=jnp.float32)
         + b0_ref[...])
    u = lax.bitcast_convert_type(o, jnp.int32)
    packed = ((_rtne16(u[:, :HALF]) & 0xFFFF)
              | (_rtne16(u[:, HALF:]) << 16))
    tn_half = packed.shape[0] // 2
    o_ref[...] = jnp.concatenate(
        [packed[:tn_half, :], packed[tn_half:, :]], axis=1)


def _node_embed(s, W0, b0):
    TN = 2000
    return pl.pallas_call(
        _embed_body,
        grid=(N_NODES // TN,),
        in_specs=[
            pl.BlockSpec((TN, C_N), lambda i: (i, 0)),
            pl.BlockSpec((C_N, BIAS), lambda i: (0, 0)),
            pl.BlockSpec((1, BIAS), lambda i: (0, 0)),
        ],
        out_specs=pl.BlockSpec((TN // 2, BIAS), lambda i: (i, 0)),
        out_shape=jax.ShapeDtypeStruct((N_NODES // 2, BIAS), jnp.int32),
    )(s, W0, b0.reshape(1, BIAS))


def _view_idx(idx):
    b = idx // 2000
    r = idx % 2000
    return b * 2000 + jnp.where(r < 1000, 2 * r, 2 * (r - 1000) + 1)


def _gather_part(node_view, idx_flat, start, size):
    info = plsc.get_sparse_core_info()
    nw = info.num_cores * info.num_subcores
    half = size // CH
    cpw = (half + nw - 1) // nw
    src_base = start
    dst_base = N_EDGES + start
    mesh = plsc.VectorSubcoreMesh(core_axis_name="c", subcore_axis_name="s")

    @functools.partial(
        pl.kernel,
        mesh=mesh,
        compiler_params=pltpu.CompilerParams(use_tc_tiling_on_sc=False),
        out_type=jax.ShapeDtypeStruct((size, 2 * HALF), jnp.int32),
        scratch_types=[
            pltpu.VMEM((CH,), jnp.int32),
            pltpu.VMEM((CH, HALF), jnp.int32),
            pltpu.SemaphoreType.DMA,
        ],
    )
    def gk(node_hbm, idx_hbm, out_hbm, idx_v, rows_v, sem):
        w = lax.axis_index("s") * info.num_cores + lax.axis_index("c")

        def make_body(idx_base, lane):
            def body(t, carry):
                cid = w * cpw + t

                @pl.when(cid < half)
                def _():
                    row_off = cid * CH
                    pltpu.sync_copy(
                        idx_hbm.at[pl.ds(idx_base + row_off, CH)], idx_v)
                    pltpu.async_copy(node_hbm.at[idx_v], rows_v, sem).wait()
                    pltpu.sync_copy(
                        rows_v,
                        out_hbm.at[pl.ds(row_off, CH), pl.ds(lane, HALF)])

                return carry
            return body

        lax.fori_loop(0, cpw, make_body(src_base, 0), 0)
        lax.fori_loop(0, cpw, make_body(dst_base, HALF), 0)

    return gk(node_view, idx_flat)


def _mlp_body(z_ref, g_ref, w1_ref, b1_ref, w2_ref, b2_ref,
              wf_ref, bf_ref, ga_ref, be_ref, prev_ref, o_ref):
    del prev_ref
    x = g_ref[...]
    lo = lax.bitcast_convert_type(x << 16, jnp.float32)
    hi = lax.bitcast_convert_type(x & jnp.int32(-65536), jnp.float32)
    e = jnp.concatenate([z_ref[...], lo, hi], axis=1)
    h = jnp.maximum(
        jnp.dot(e, w1_ref[...], preferred_element_type=jnp.float32)
        + b1_ref[...], 0.0)
    h = jnp.maximum(
        jnp.dot(h, w2_ref[...], preferred_element_type=jnp.float32)
        + b2_ref[...], 0.0)
    o = (jnp.dot(h + e, wf_ref[...], preferred_element_type=jnp.float32)
         + bf_ref[...])
    mu = jnp.mean(o, axis=1, keepdims=True)
    c = o - mu
    var = jnp.mean(c * c, axis=1, keepdims=True)
    o_ref[...] = c * lax.rsqrt(var + 1e-5) * ga_ref[...] + be_ref[...]


def _mlp_body_first(z_ref, g_ref, w1_ref, b1_ref, w2_ref, b2_ref,
                    wf_ref, bf_ref, ga_ref, be_ref, o_ref):
    _mlp_body(z_ref, g_ref, w1_ref, b1_ref, w2_ref, b2_ref,
              wf_ref, bf_ref, ga_ref, be_ref, None, o_ref)


def _mlp_part(z, gathered, weights, prev_out, start, size):
    W1p, b1, W2, b2, Wfp, bf, gamma, beta = weights
    goff = start // TE
    gp = size // TE

    def _const2(shape):
        return pl.BlockSpec(shape, lambda i: (0, 0))

    in_specs = [
        pl.BlockSpec((TE, C_Z), lambda i: (i + goff, 0)),
        pl.BlockSpec((TE, BIAS), lambda i: (i, 0)),
        _const2((HID, HID)),
        _const2((1, HID)),
        _const2((HID, HID)),
        _const2((1, HID)),
        _const2((HID, C_Z)),
        _const2((1, C_Z)),
        _const2((1, C_Z)),
        _const2((1, C_Z)),
    ]
    args = [z, gathered, W1p, b1.reshape(1, HID), W2,
            b2.reshape(1, HID), Wfp, bf.reshape(1, C_Z),
            gamma.reshape(1, C_Z), beta.reshape(1, C_Z)]
    if prev_out is None:
        body = _mlp_body_first
        aliases = {}
    else:
        body = _mlp_body
        in_specs.append(pl.BlockSpec(memory_space=pl.ANY))
        args.append(prev_out)
        aliases = {10: 0}

    return pl.pallas_call(
        body,
        grid=(gp,),
        in_specs=in_specs,
        out_specs=pl.BlockSpec((TE, C_Z), lambda i: (i + goff, 0)),
        out_shape=jax.ShapeDtypeStruct((N_EDGES, C_Z), jnp.float32),
        input_output_aliases=aliases,
    )(*args)


def kernel(s, z, W0, b0, W1, b1, W2, b2, Wf, bf, gamma, beta, edge_index):
    node_packed = _node_embed(s, W0, b0)
    node_view = node_packed.reshape(N_NODES, HALF)
    idx_flat = _view_idx(edge_index.reshape(-1))
    perm = jnp.asarray(_PERM)
    weights = (W1[perm, :], b1, W2[:, perm], b2[perm], Wf[perm, :], bf,
               gamma, beta)

    starts = [sum(PARTS[:p]) for p in range(len(PARTS))]
    gathered = [_gather_part(node_view, idx_flat, st, sz)
                for st, sz in zip(starts, PARTS)]
    out = None
    for g, st, sz in zip(gathered, starts, PARTS):
        out = _mlp_part(z, g, weights, out, st, sz)
    return out


# speedup vs baseline: 1.0149x; 1.0149x over previous
import functools

import jax
import jax.numpy as jnp
import numpy as np
from jax import lax
from jax.experimental import pallas as pl
from jax.experimental.pallas import tpu as pltpu
from jax.experimental.pallas import tpu_sc as plsc

N_NODES = 10000
N_EDGES = 160000
C_N = 256
C_Z = 128
BIAS = 128
HALF = BIAS // 2
HID = 384

P = 5
EP = N_EDGES // P
TE = 1600
GP = EP // TE
CH = 128

_PERM = np.concatenate([
    np.arange(0, 128),
    np.arange(128, 192),
    np.arange(256, 320),
    np.arange(192, 256),
    np.arange(320, 384),
])


def _rtne16(u):
    return (u + 0x7FFF + ((u >> 16) & 1)) >> 16


def _embed_body(s_ref, w0_ref, b0_ref, ei_ref, o_ref, oi_ref):
    o = (jnp.dot(s_ref[...], w0_ref[...], preferred_element_type=jnp.float32)
         + b0_ref[...])
    u = lax.bitcast_convert_type(o, jnp.int32)
    packed = ((_rtne16(u[:, :HALF]) & 0xFFFF)
              | (_rtne16(u[:, HALF:]) << 16))
    tn_half = packed.shape[0] // 2
    o_ref[...] = jnp.concatenate(
        [packed[:tn_half, :], packed[tn_half:, :]], axis=1)
    n = ei_ref[...]
    b = n // 2000
    r = n - b * 2000
    oi_ref[...] = b * 2000 + 2 * r - jnp.where(r < 1000, 0, 1999)


def _node_embed(s, W0, b0, edge_index):
    TN = 2000
    ei8 = edge_index.reshape(8, 2 * N_EDGES // 8)
    ib = 2 * N_EDGES // 8
    return pl.pallas_call(
        _embed_body,
        grid=(N_NODES // TN,),
        in_specs=[
            pl.BlockSpec((TN, C_N), lambda i: (i, 0)),
            pl.BlockSpec((C_N, BIAS), lambda i: (0, 0)),
            pl.BlockSpec((1, BIAS), lambda i: (0, 0)),
            pl.BlockSpec((8, ib), lambda i: (0, 0)),
        ],
        out_specs=[
            pl.BlockSpec((TN // 2, BIAS), lambda i: (i, 0)),
            pl.BlockSpec((8, ib), lambda i: (0, 0)),
        ],
        out_shape=[
            jax.ShapeDtypeStruct((N_NODES // 2, BIAS), jnp.int32),
            jax.ShapeDtypeStruct((8, 2 * N_EDGES // 8), jnp.int32),
        ],
    )(s, W0, b0.reshape(1, BIAS), ei8)


def _gather_part(node_view, idx_flat, part):
    info = plsc.get_sparse_core_info()
    nw = info.num_cores * info.num_subcores
    half = EP // CH
    cpw = (half + nw - 1) // nw
    src_base = part * EP
    dst_base = N_EDGES + part * EP
    mesh = plsc.VectorSubcoreMesh(core_axis_name="c", subcore_axis_name="s")

    @functools.partial(
        pl.kernel,
        mesh=mesh,
        compiler_params=pltpu.CompilerParams(use_tc_tiling_on_sc=False),
        out_type=jax.ShapeDtypeStruct((EP, 2 * HALF), jnp.int32),
        scratch_types=[
            pltpu.VMEM((CH,), jnp.int32),
            pltpu.VMEM((CH, HALF), jnp.int32),
            pltpu.SemaphoreType.DMA,
        ],
    )
    def gk(node_hbm, idx_hbm, out_hbm, idx_v, rows_v, sem):
        w = lax.axis_index("s") * info.num_cores + lax.axis_index("c")

        def make_body(idx_base, lane):
            def body(t, carry):
                cid = w * cpw + t

                @pl.when(cid < half)
                def _():
                    row_off = cid * CH
                    pltpu.sync_copy(
                        idx_hbm.at[pl.ds(idx_base + row_off, CH)], idx_v)
                    pltpu.async_copy(node_hbm.at[idx_v], rows_v, sem).wait()
                    pltpu.sync_copy(
                        rows_v,
                        out_hbm.at[pl.ds(row_off, CH), pl.ds(lane, HALF)])

                return carry
            return body

        lax.fori_loop(0, cpw, make_body(src_base, 0), 0)
        lax.fori_loop(0, cpw, make_body(dst_base, HALF), 0)

    return gk(node_view, idx_flat)


def _mlp_body(z_ref, g_ref, w1_ref, b1_ref, w2_ref, b2_ref,
              wf_ref, bf_ref, ga_ref, be_ref, prev_ref, o_ref):
    del prev_ref
    x = g_ref[...]
    lo = lax.bitcast_convert_type(x << 16, jnp.float32)
    hi = lax.bitcast_convert_type(x & jnp.int32(-65536), jnp.float32)
    e = jnp.concatenate([z_ref[...], lo, hi], axis=1)
    h = jnp.maximum(
        jnp.dot(e, w1_ref[...], preferred_element_type=jnp.float32)
        + b1_ref[...], 0.0)
    h = jnp.maximum(
        jnp.dot(h, w2_ref[...], preferred_element_type=jnp.float32)
        + b2_ref[...], 0.0)
    o = (jnp.dot(h + e, wf_ref[...], preferred_element_type=jnp.float32)
         + bf_ref[...])
    mu = jnp.mean(o, axis=1, keepdims=True)
    c = o - mu
    var = jnp.mean(c * c, axis=1, keepdims=True)
    o_ref[...] = c * lax.rsqrt(var + 1e-5) * ga_ref[...] + be_ref[...]


def _mlp_body_first(z_ref, g_ref, w1_ref, b1_ref, w2_ref, b2_ref,
                    wf_ref, bf_ref, ga_ref, be_ref, o_ref):
    _mlp_body(z_ref, g_ref, w1_ref, b1_ref, w2_ref, b2_ref,
              wf_ref, bf_ref, ga_ref, be_ref, None, o_ref)


def _mlp_part(z, gathered, weights, prev_out, part):
    W1p, b1, W2, b2, Wfp, bf, gamma, beta = weights

    def _const2(shape):
        return pl.BlockSpec(shape, lambda i: (0, 0))

    in_specs = [
        pl.BlockSpec((TE, C_Z), lambda i: (i + part * GP, 0)),
        pl.BlockSpec((TE, BIAS), lambda i: (i, 0)),
        _const2((HID, HID)),
        _const2((1, HID)),
        _const2((HID, HID)),
        _const2((1, HID)),
        _const2((HID, C_Z)),
        _const2((1, C_Z)),
        _const2((1, C_Z)),
        _const2((1, C_Z)),
    ]
    args = [z, gathered, W1p, b1.reshape(1, HID), W2,
            b2.reshape(1, HID), Wfp, bf.reshape(1, C_Z),
            gamma.reshape(1, C_Z), beta.reshape(1, C_Z)]
    if prev_out is None:
        body = _mlp_body_first
        aliases = {}
    else:
        body = _mlp_body
        in_specs.append(pl.BlockSpec(memory_space=pl.ANY))
        args.append(prev_out)
        aliases = {10: 0}

    return pl.pallas_call(
        body,
        grid=(GP,),
        in_specs=in_specs,
        out_specs=pl.BlockSpec((TE, C_Z), lambda i: (i + part * GP, 0)),
        out_shape=jax.ShapeDtypeStruct((N_EDGES, C_Z), jnp.float32),
        input_output_aliases=aliases,
    )(*args)


def kernel(s, z, W0, b0, W1, b1, W2, b2, Wf, bf, gamma, beta, edge_index):
    node_packed, idx8 = _node_embed(s, W0, b0, edge_index)
    node_view = node_packed.reshape(N_NODES, HALF)
    idx_flat = idx8.reshape(-1)
    perm = jnp.asarray(_PERM)
    weights = (W1[perm, :], b1, W2[:, perm], b2[perm], Wf[perm, :], bf,
               gamma, beta)

    gathered = [_gather_part(node_view, idx_flat, p) for p in range(P)]
    out = None
    for p in range(P):
        out = _mlp_part(z, gathered[p], weights, out, p)
    return out


# speedup vs baseline: 1.0585x; 1.0429x over previous
import functools

import jax
import jax.numpy as jnp
import numpy as np
from jax import lax
from jax.experimental import pallas as pl
from jax.experimental.pallas import tpu as pltpu
from jax.experimental.pallas import tpu_sc as plsc

N_NODES = 10000
N_EDGES = 160000
C_N = 256
C_Z = 128
BIAS = 128
HALF = BIAS // 2
HID = 384

P = 5
EP = N_EDGES // P
TE = 1600
GP = EP // TE
CH = 128

_PERM = np.concatenate([
    np.arange(0, 128),
    np.arange(128, 192),
    np.arange(256, 320),
    np.arange(192, 256),
    np.arange(320, 384),
])


def _rtne16(u):
    return (u + 0x7FFF + ((u >> 16) & 1)) >> 16


def _embed_body(s_ref, w0_ref, b0_ref, o_ref):
    o = (jnp.dot(s_ref[...], w0_ref[...], preferred_element_type=jnp.float32)
         + b0_ref[...])
    u = lax.bitcast_convert_type(o, jnp.int32)
    packed = ((_rtne16(u[:, :HALF]) & 0xFFFF)
              | (_rtne16(u[:, HALF:]) << 16))
    tn_half = packed.shape[0] // 2
    o_ref[...] = jnp.concatenate(
        [packed[:tn_half, :], packed[tn_half:, :]], axis=1)


def _node_embed(s, W0, b0):
    TN = 2000
    return pl.pallas_call(
        _embed_body,
        grid=(N_NODES // TN,),
        in_specs=[
            pl.BlockSpec((TN, C_N), lambda i: (i, 0)),
            pl.BlockSpec((C_N, BIAS), lambda i: (0, 0)),
            pl.BlockSpec((1, BIAS), lambda i: (0, 0)),
        ],
        out_specs=pl.BlockSpec((TN // 2, BIAS), lambda i: (i, 0)),
        out_shape=jax.ShapeDtypeStruct((N_NODES // 2, BIAS), jnp.int32),
    )(s, W0, b0.reshape(1, BIAS))


def _view_idx(idx):
    b = idx // 2000
    r = idx % 2000
    return b * 2000 + jnp.where(r < 1000, 2 * r, 2 * (r - 1000) + 1)


def _gather_part(node_view, idx_flat, part):
    info = plsc.get_sparse_core_info()
    nw = info.num_cores * info.num_subcores
    half = EP // CH
    cpw = (half + nw - 1) // nw
    src_base = part * EP
    dst_base = N_EDGES + part * EP
    mesh = plsc.VectorSubcoreMesh(core_axis_name="c", subcore_axis_name="s")

    @functools.partial(
        pl.kernel,
        mesh=mesh,
        compiler_params=pltpu.CompilerParams(use_tc_tiling_on_sc=False),
        out_type=jax.ShapeDtypeStruct((EP, 2 * HALF), jnp.int32),
        scratch_types=[
            pltpu.VMEM((CH,), jnp.int32),
            pltpu.VMEM((CH,), jnp.int32),
            pltpu.VMEM((CH, HALF), jnp.int32),
            pltpu.VMEM((CH, HALF), jnp.int32),
            pltpu.SemaphoreType.DMA,
            pltpu.SemaphoreType.DMA,
        ],
    )
    def gk(node_hbm, idx_hbm, out_hbm, idx_v0, idx_v1, rows_v0, rows_v1,
           sem0, sem1):
        w = lax.axis_index("s") * info.num_cores + lax.axis_index("c")

        def issue(cid, idx_base, idx_v, rows_v, sem):
            @pl.when(cid < half)
            def _():
                pltpu.sync_copy(
                    idx_hbm.at[pl.ds(idx_base + cid * CH, CH)], idx_v)
                pltpu.async_copy(node_hbm.at[idx_v], rows_v, sem)

        def drain(cid, lane, idx_v, rows_v, sem):
            @pl.when(cid < half)
            def _():
                pltpu.make_async_copy(
                    node_hbm.at[idx_v], rows_v, sem).wait()
                pltpu.sync_copy(
                    rows_v,
                    out_hbm.at[pl.ds(cid * CH, CH), pl.ds(lane, HALF)])

        issue(w * cpw, src_base, idx_v0, rows_v0, sem0)

        def body(t, carry):
            cid = w * cpw + t
            issue(cid, dst_base, idx_v1, rows_v1, sem1)
            drain(cid, 0, idx_v0, rows_v0, sem0)

            @pl.when(t + 1 < cpw)
            def _():
                issue(cid + 1, src_base, idx_v0, rows_v0, sem0)

            drain(cid, HALF, idx_v1, rows_v1, sem1)
            return carry

        lax.fori_loop(0, cpw, body, 0)

    return gk(node_view, idx_flat)


def _mlp_body(z_ref, g_ref, w1_ref, b1_ref, w2_ref, b2_ref,
              wf_ref, bf_ref, ga_ref, be_ref, prev_ref, o_ref):
    del prev_ref
    x = g_ref[...]
    lo = lax.bitcast_convert_type(x << 16, jnp.float32)
    hi = lax.bitcast_convert_type(x & jnp.int32(-65536), jnp.float32)
    e = jnp.concatenate([z_ref[...], lo, hi], axis=1)
    h = jnp.maximum(
        jnp.dot(e, w1_ref[...], preferred_element_type=jnp.float32)
        + b1_ref[...], 0.0)
    h = jnp.maximum(
        jnp.dot(h, w2_ref[...], preferred_element_type=jnp.float32)
        + b2_ref[...], 0.0)
    o = (jnp.dot(h + e, wf_ref[...], preferred_element_type=jnp.float32)
         + bf_ref[...])
    mu = jnp.mean(o, axis=1, keepdims=True)
    c = o - mu
    var = jnp.mean(c * c, axis=1, keepdims=True)
    o_ref[...] = c * lax.rsqrt(var + 1e-5) * ga_ref[...] + be_ref[...]


def _mlp_body_first(z_ref, g_ref, w1_ref, b1_ref, w2_ref, b2_ref,
                    wf_ref, bf_ref, ga_ref, be_ref, o_ref):
    _mlp_body(z_ref, g_ref, w1_ref, b1_ref, w2_ref, b2_ref,
              wf_ref, bf_ref, ga_ref, be_ref, None, o_ref)


def _mlp_part(z, gathered, weights, prev_out, part):
    W1p, b1, W2, b2, Wfp, bf, gamma, beta = weights

    def _const2(shape):
        return pl.BlockSpec(shape, lambda i: (0, 0))

    in_specs = [
        pl.BlockSpec((TE, C_Z), lambda i: (i + part * GP, 0)),
        pl.BlockSpec((TE, BIAS), lambda i: (i, 0)),
        _const2((HID, HID)),
        _const2((1, HID)),
        _const2((HID, HID)),
        _const2((1, HID)),
        _const2((HID, C_Z)),
        _const2((1, C_Z)),
        _const2((1, C_Z)),
        _const2((1, C_Z)),
    ]
    args = [z, gathered, W1p, b1.reshape(1, HID), W2,
            b2.reshape(1, HID), Wfp, bf.reshape(1, C_Z),
            gamma.reshape(1, C_Z), beta.reshape(1, C_Z)]
    if prev_out is None:
        body = _mlp_body_first
        aliases = {}
    else:
        body = _mlp_body
        in_specs.append(pl.BlockSpec(memory_space=pl.ANY))
        args.append(prev_out)
        aliases = {10: 0}

    return pl.pallas_call(
        body,
        grid=(GP,),
        in_specs=in_specs,
        out_specs=pl.BlockSpec((TE, C_Z), lambda i: (i + part * GP, 0)),
        out_shape=jax.ShapeDtypeStruct((N_EDGES, C_Z), jnp.float32),
        input_output_aliases=aliases,
    )(*args)


def kernel(s, z, W0, b0, W1, b1, W2, b2, Wf, bf, gamma, beta, edge_index):
    node_packed = _node_embed(s, W0, b0)
    node_view = node_packed.reshape(N_NODES, HALF)
    idx_flat = _view_idx(edge_index.reshape(-1))
    perm = jnp.asarray(_PERM)
    weights = (W1[perm, :], b1, W2[:, perm], b2[perm], Wf[perm, :], bf,
               gamma, beta)

    gathered = [_gather_part(node_view, idx_flat, p) for p in range(P)]
    out = None
    for p in range(P):
        out = _mlp_part(z, gathered[p], weights, out, p)
    return out


# speedup vs baseline: 1.0727x; 1.0135x over previous
import functools

import jax
import jax.numpy as jnp
import numpy as np
from jax import lax
from jax.experimental import pallas as pl
from jax.experimental.pallas import tpu as pltpu
from jax.experimental.pallas import tpu_sc as plsc

N_NODES = 10000
N_EDGES = 160000
C_N = 256
C_Z = 128
BIAS = 128
HALF = BIAS // 2
HID = 384

P = 5
EP = N_EDGES // P
TE = 2000
GP = EP // TE
CH = 128

_PERM = np.concatenate([
    np.arange(0, 128),
    np.arange(128, 192),
    np.arange(256, 320),
    np.arange(192, 256),
    np.arange(320, 384),
])


def _rtne16(u):
    return (u + 0x7FFF + ((u >> 16) & 1)) >> 16


def _embed_body(s_ref, w0_ref, b0_ref, o_ref):
    o = (jnp.dot(s_ref[...], w0_ref[...], preferred_element_type=jnp.float32)
         + b0_ref[...])
    u = lax.bitcast_convert_type(o, jnp.int32)
    packed = ((_rtne16(u[:, :HALF]) & 0xFFFF)
              | (_rtne16(u[:, HALF:]) << 16))
    tn_half = packed.shape[0] // 2
    o_ref[...] = jnp.concatenate(
        [packed[:tn_half, :], packed[tn_half:, :]], axis=1)


def _node_embed(s, W0, b0):
    TN = 2000
    return pl.pallas_call(
        _embed_body,
        grid=(N_NODES // TN,),
        in_specs=[
            pl.BlockSpec((TN, C_N), lambda i: (i, 0)),
            pl.BlockSpec((C_N, BIAS), lambda i: (0, 0)),
            pl.BlockSpec((1, BIAS), lambda i: (0, 0)),
        ],
        out_specs=pl.BlockSpec((TN // 2, BIAS), lambda i: (i, 0)),
        out_shape=jax.ShapeDtypeStruct((N_NODES // 2, BIAS), jnp.int32),
    )(s, W0, b0.reshape(1, BIAS))


def _view_idx(idx):
    b = idx // 2000
    r = idx % 2000
    return b * 2000 + jnp.where(r < 1000, 2 * r, 2 * (r - 1000) + 1)


def _gather_part(node_view, idx_flat, part):
    info = plsc.get_sparse_core_info()
    nw = info.num_cores * info.num_subcores
    half = EP // CH
    cpw = (half + nw - 1) // nw
    src_base = part * EP
    dst_base = N_EDGES + part * EP
    mesh = plsc.VectorSubcoreMesh(core_axis_name="c", subcore_axis_name="s")

    @functools.partial(
        pl.kernel,
        mesh=mesh,
        compiler_params=pltpu.CompilerParams(use_tc_tiling_on_sc=False),
        out_type=jax.ShapeDtypeStruct((EP, 2 * HALF), jnp.int32),
        scratch_types=[
            pltpu.VMEM((CH,), jnp.int32),
            pltpu.VMEM((CH,), jnp.int32),
            pltpu.VMEM((CH, HALF), jnp.int32),
            pltpu.VMEM((CH, HALF), jnp.int32),
            pltpu.SemaphoreType.DMA,
            pltpu.SemaphoreType.DMA,
        ],
    )
    def gk(node_hbm, idx_hbm, out_hbm, idx_v0, idx_v1, rows_v0, rows_v1,
           sem0, sem1):
        w = lax.axis_index("s") * info.num_cores + lax.axis_index("c")

        def issue(cid, idx_base, idx_v, rows_v, sem):
            @pl.when(cid < half)
            def _():
                pltpu.sync_copy(
                    idx_hbm.at[pl.ds(idx_base + cid * CH, CH)], idx_v)
                pltpu.async_copy(node_hbm.at[idx_v], rows_v, sem)

        def drain(cid, lane, idx_v, rows_v, sem):
            @pl.when(cid < half)
            def _():
                pltpu.make_async_copy(
                    node_hbm.at[idx_v], rows_v, sem).wait()
                pltpu.sync_copy(
                    rows_v,
                    out_hbm.at[pl.ds(cid * CH, CH), pl.ds(lane, HALF)])

        issue(w * cpw, src_base, idx_v0, rows_v0, sem0)

        def body(t, carry):
            cid = w * cpw + t
            issue(cid, dst_base, idx_v1, rows_v1, sem1)
            drain(cid, 0, idx_v0, rows_v0, sem0)

            @pl.when(t + 1 < cpw)
            def _():
                issue(cid + 1, src_base, idx_v0, rows_v0, sem0)

            drain(cid, HALF, idx_v1, rows_v1, sem1)
            return carry

        lax.fori_loop(0, cpw, body, 0)

    return gk(node_view, idx_flat)


def _mlp_body(z_ref, g_ref, w1_ref, b1_ref, w2_ref, b2_ref,
              wf_ref, bf_ref, ga_ref, be_ref, prev_ref, o_ref):
    del prev_ref
    x = g_ref[...]
    lo = lax.bitcast_convert_type(x << 16, jnp.float32)
    hi = lax.bitcast_convert_type(x & jnp.int32(-65536), jnp.float32)
    e = jnp.concatenate([z_ref[...], lo, hi], axis=1)
    h = jnp.maximum(
        jnp.dot(e, w1_ref[...], preferred_element_type=jnp.float32)
        + b1_ref[...], 0.0)
    h = jnp.maximum(
        jnp.dot(h, w2_ref[...], preferred_element_type=jnp.float32)
        + b2_ref[...], 0.0)
    o = (jnp.dot(h + e, wf_ref[...], preferred_element_type=jnp.float32)
         + bf_ref[...])
    mu = jnp.mean(o, axis=1, keepdims=True)
    c = o - mu
    var = jnp.mean(c * c, axis=1, keepdims=True)
    o_ref[...] = c * lax.rsqrt(var + 1e-5) * ga_ref[...] + be_ref[...]


def _mlp_body_first(z_ref, g_ref, w1_ref, b1_ref, w2_ref, b2_ref,
                    wf_ref, bf_ref, ga_ref, be_ref, o_ref):
    _mlp_body(z_ref, g_ref, w1_ref, b1_ref, w2_ref, b2_ref,
              wf_ref, bf_ref, ga_ref, be_ref, None, o_ref)


def _mlp_part(z, gathered, weights, prev_out, part):
    W1p, b1, W2, b2, Wfp, bf, gamma, beta = weights

    def _const2(shape):
        return pl.BlockSpec(shape, lambda i: (0, 0))

    in_specs = [
        pl.BlockSpec((TE, C_Z), lambda i: (i + part * GP, 0)),
        pl.BlockSpec((TE, BIAS), lambda i: (i, 0)),
        _const2((HID, HID)),
        _const2((1, HID)),
        _const2((HID, HID)),
        _const2((1, HID)),
        _const2((HID, C_Z)),
        _const2((1, C_Z)),
        _const2((1, C_Z)),
        _const2((1, C_Z)),
    ]
    args = [z, gathered, W1p, b1.reshape(1, HID), W2,
            b2.reshape(1, HID), Wfp, bf.reshape(1, C_Z),
            gamma.reshape(1, C_Z), beta.reshape(1, C_Z)]
    if prev_out is None:
        body = _mlp_body_first
        aliases = {}
    else:
        body = _mlp_body
        in_specs.append(pl.BlockSpec(memory_space=pl.ANY))
        args.append(prev_out)
        aliases = {10: 0}

    return pl.pallas_call(
        body,
        grid=(GP,),
        in_specs=in_specs,
        out_specs=pl.BlockSpec((TE, C_Z), lambda i: (i + part * GP, 0)),
        out_shape=jax.ShapeDtypeStruct((N_EDGES, C_Z), jnp.float32),
        input_output_aliases=aliases,
    )(*args)


def kernel(s, z, W0, b0, W1, b1, W2, b2, Wf, bf, gamma, beta, edge_index):
    node_packed = _node_embed(s, W0, b0)
    node_view = node_packed.reshape(N_NODES, HALF)
    idx_flat = _view_idx(edge_index.reshape(-1))
    perm = jnp.asarray(_PERM)
    weights = (W1[perm, :], b1, W2[:, perm], b2[perm], Wf[perm, :], bf,
               gamma, beta)

    gathered = [_gather_part(node_view, idx_flat, p) for p in range(P)]
    out = None
    for p in range(P):
        out = _mlp_part(z, gathered[p], weights, out, p)
    return out


# speedup vs baseline: 1.1122x; 1.0368x over previous
import functools

import jax
import jax.numpy as jnp
import numpy as np
from jax import lax
from jax.experimental import pallas as pl
from jax.experimental.pallas import tpu as pltpu
from jax.experimental.pallas import tpu_sc as plsc

N_NODES = 10000
N_EDGES = 160000
C_N = 256
C_Z = 128
BIAS = 128
HALF = BIAS // 2
HID = 384

PARTS = (16000, 32000, 48000, 64000)
TE = 2000
CH = 128

_PERM = np.concatenate([
    np.arange(0, 128),
    np.arange(128, 192),
    np.arange(256, 320),
    np.arange(192, 256),
    np.arange(320, 384),
])


def _rtne16(u):
    return (u + 0x7FFF + ((u >> 16) & 1)) >> 16


def _embed_body(s_ref, w0_ref, b0_ref, o_ref):
    o = (jnp.dot(s_ref[...], w0_ref[...], preferred_element_type=jnp.float32)
         + b0_ref[...])
    u = lax.bitcast_convert_type(o, jnp.int32)
    packed = ((_rtne16(u[:, :HALF]) & 0xFFFF)
              | (_rtne16(u[:, HALF:]) << 16))
    tn_half = packed.shape[0] // 2
    o_ref[...] = jnp.concatenate(
        [packed[:tn_half, :], packed[tn_half:, :]], axis=1)


def _node_embed(s, W0, b0):
    TN = 2000
    return pl.pallas_call(
        _embed_body,
        grid=(N_NODES // TN,),
        in_specs=[
            pl.BlockSpec((TN, C_N), lambda i: (i, 0)),
            pl.BlockSpec((C_N, BIAS), lambda i: (0, 0)),
            pl.BlockSpec((1, BIAS), lambda i: (0, 0)),
        ],
        out_specs=pl.BlockSpec((TN // 2, BIAS), lambda i: (i, 0)),
        out_shape=jax.ShapeDtypeStruct((N_NODES // 2, BIAS), jnp.int32),
    )(s, W0, b0.reshape(1, BIAS))


def _view_idx(idx):
    b = idx // 2000
    r = idx % 2000
    return b * 2000 + jnp.where(r < 1000, 2 * r, 2 * (r - 1000) + 1)


def _gather_part(node_view, idx_flat, start, size):
    info = plsc.get_sparse_core_info()
    nw = info.num_cores * info.num_subcores
    half = size // CH
    cpw = (half + nw - 1) // nw
    src_base = start
    dst_base = N_EDGES + start
    mesh = plsc.VectorSubcoreMesh(core_axis_name="c", subcore_axis_name="s")

    @functools.partial(
        pl.kernel,
        mesh=mesh,
        compiler_params=pltpu.CompilerParams(use_tc_tiling_on_sc=False),
        out_type=jax.ShapeDtypeStruct((size, 2 * HALF), jnp.int32),
        scratch_types=[
            pltpu.VMEM((CH,), jnp.int32),
            pltpu.VMEM((CH,), jnp.int32),
            pltpu.VMEM((CH, HALF), jnp.int32),
            pltpu.VMEM((CH, HALF), jnp.int32),
            pltpu.SemaphoreType.DMA,
            pltpu.SemaphoreType.DMA,
        ],
    )
    def gk(node_hbm, idx_hbm, out_hbm, idx_v0, idx_v1, rows_v0, rows_v1,
           sem0, sem1):
        w = lax.axis_index("s") * info.num_cores + lax.axis_index("c")

        def issue(cid, idx_base, idx_v, rows_v, sem):
            @pl.when(cid < half)
            def _():
                pltpu.sync_copy(
                    idx_hbm.at[pl.ds(idx_base + cid * CH, CH)], idx_v)
                pltpu.async_copy(node_hbm.at[idx_v], rows_v, sem)

        def drain(cid, lane, idx_v, rows_v, sem):
            @pl.when(cid < half)
            def _():
                pltpu.make_async_copy(
                    node_hbm.at[idx_v], rows_v, sem).wait()
                pltpu.sync_copy(
                    rows_v,
                    out_hbm.at[pl.ds(cid * CH, CH), pl.ds(lane, HALF)])

        issue(w * cpw, src_base, idx_v0, rows_v0, sem0)

        def body(t, carry):
            cid = w * cpw + t
            issue(cid, dst_base, idx_v1, rows_v1, sem1)
            drain(cid, 0, idx_v0, rows_v0, sem0)

            @pl.when(t + 1 < cpw)
            def _():
                issue(cid + 1, src_base, idx_v0, rows_v0, sem0)

            drain(cid, HALF, idx_v1, rows_v1, sem1)
            return carry

        lax.fori_loop(0, cpw, body, 0)

    return gk(node_view, idx_flat)


def _mlp_body(z_ref, g_ref, w1_ref, b1_ref, w2_ref, b2_ref,
              wf_ref, bf_ref, ga_ref, be_ref, prev_ref, o_ref):
    del prev_ref
    x = g_ref[...]
    lo = lax.bitcast_convert_type(x << 16, jnp.float32)
    hi = lax.bitcast_convert_type(x & jnp.int32(-65536), jnp.float32)
    e = jnp.concatenate([z_ref[...], lo, hi], axis=1)
    h = jnp.maximum(
        jnp.dot(e, w1_ref[...], preferred_element_type=jnp.float32)
        + b1_ref[...], 0.0)
    h = jnp.maximum(
        jnp.dot(h, w2_ref[...], preferred_element_type=jnp.float32)
        + b2_ref[...], 0.0)
    o = (jnp.dot(h + e, wf_ref[...], preferred_element_type=jnp.float32)
         + bf_ref[...])
    mu = jnp.mean(o, axis=1, keepdims=True)
    c = o - mu
    var = jnp.mean(c * c, axis=1, keepdims=True)
    o_ref[...] = c * lax.rsqrt(var + 1e-5) * ga_ref[...] + be_ref[...]


def _mlp_body_first(z_ref, g_ref, w1_ref, b1_ref, w2_ref, b2_ref,
                    wf_ref, bf_ref, ga_ref, be_ref, o_ref):
    _mlp_body(z_ref, g_ref, w1_ref, b1_ref, w2_ref, b2_ref,
              wf_ref, bf_ref, ga_ref, be_ref, None, o_ref)


def _mlp_part(z, gathered, weights, prev_out, start, size):
    W1p, b1, W2, b2, Wfp, bf, gamma, beta = weights
    goff = start // TE
    gp = size // TE

    def _const2(shape):
        return pl.BlockSpec(shape, lambda i: (0, 0))

    in_specs = [
        pl.BlockSpec((TE, C_Z), lambda i: (i + goff, 0)),
        pl.BlockSpec((TE, BIAS), lambda i: (i, 0)),
        _const2((HID, HID)),
        _const2((1, HID)),
        _const2((HID, HID)),
        _const2((1, HID)),
        _const2((HID, C_Z)),
        _const2((1, C_Z)),
        _const2((1, C_Z)),
        _const2((1, C_Z)),
    ]
    args = [z, gathered, W1p, b1.reshape(1, HID), W2,
            b2.reshape(1, HID), Wfp, bf.reshape(1, C_Z),
            gamma.reshape(1, C_Z), beta.reshape(1, C_Z)]
    if prev_out is None:
        body = _mlp_body_first
        aliases = {}
    else:
        body = _mlp_body
        in_specs.append(pl.BlockSpec(memory_space=pl.ANY))
        args.append(prev_out)
        aliases = {10: 0}

    return pl.pallas_call(
        body,
        grid=(gp,),
        in_specs=in_specs,
        out_specs=pl.BlockSpec((TE, C_Z), lambda i: (i + goff, 0)),
        out_shape=jax.ShapeDtypeStruct((N_EDGES, C_Z), jnp.float32),
        input_output_aliases=aliases,
    )(*args)


def kernel(s, z, W0, b0, W1, b1, W2, b2, Wf, bf, gamma, beta, edge_index):
    node_packed = _node_embed(s, W0, b0)
    node_view = node_packed.reshape(N_NODES, HALF)
    idx_flat = _view_idx(edge_index.reshape(-1))
    perm = jnp.asarray(_PERM)
    weights = (W1[perm, :], b1, W2[:, perm], b2[perm], Wf[perm, :], bf,
               gamma, beta)

    starts = [sum(PARTS[:p]) for p in range(len(PARTS))]
    gathered = [_gather_part(node_view, idx_flat, st, sz)
                for st, sz in zip(starts, PARTS)]
    out = None
    for g, st, sz in zip(gathered, starts, PARTS):
        out = _mlp_part(z, g, weights, out, st, sz)
    return out


# speedup vs baseline: 1.1123x; 1.0001x over previous
import functools

import jax
import jax.numpy as jnp
import numpy as np
from jax import lax
from jax.experimental import pallas as pl
from jax.experimental.pallas import tpu as pltpu
from jax.experimental.pallas import tpu_sc as plsc

N_NODES = 10000
N_EDGES = 160000
C_N = 256
C_Z = 128
BIAS = 128
HALF = BIAS // 2
HID = 384

PARTS = (16000, 32000, 48000, 64000)
TE = 2000
CH = 128

_PERM = np.concatenate([
    np.arange(0, 128),
    np.arange(128, 192),
    np.arange(256, 320),
    np.arange(192, 256),
    np.arange(320, 384),
])


def _rtne16(u):
    return (u + 0x7FFF + ((u >> 16) & 1)) >> 16


def _embed_body(s_ref, w0_ref, b0_ref, o_ref):
    o = (jnp.dot(s_ref[...], w0_ref[...], preferred_element_type=jnp.float32)
         + b0_ref[...])
    u = lax.bitcast_convert_type(o, jnp.int32)
    packed = ((_rtne16(u[:, :HALF]) & 0xFFFF)
              | (_rtne16(u[:, HALF:]) << 16))
    tn_half = packed.shape[0] // 2
    o_ref[...] = jnp.concatenate(
        [packed[:tn_half, :], packed[tn_half:, :]], axis=1)


def _node_embed(s, W0, b0):
    TN = 2000
    return pl.pallas_call(
        _embed_body,
        grid=(N_NODES // TN,),
        in_specs=[
            pl.BlockSpec((TN, C_N), lambda i: (i, 0)),
            pl.BlockSpec((C_N, BIAS), lambda i: (0, 0)),
            pl.BlockSpec((1, BIAS), lambda i: (0, 0)),
        ],
        out_specs=pl.BlockSpec((TN // 2, BIAS), lambda i: (i, 0)),
        out_shape=jax.ShapeDtypeStruct((N_NODES // 2, BIAS), jnp.int32),
    )(s, W0, b0.reshape(1, BIAS))


def _view_idx(idx):
    b = idx // 2000
    r = idx % 2000
    return b * 2000 + jnp.where(r < 1000, 2 * r, 2 * (r - 1000) + 1)


def _gather_part(node_view, idx_flat, start, size):
    info = plsc.get_sparse_core_info()
    nw = info.num_cores * info.num_subcores
    half = size // CH
    cpw = (half + nw - 1) // nw
    src_base = start
    dst_base = N_EDGES + start
    mesh = plsc.VectorSubcoreMesh(core_axis_name="c", subcore_axis_name="s")

    @functools.partial(
        pl.kernel,
        mesh=mesh,
        compiler_params=pltpu.CompilerParams(use_tc_tiling_on_sc=False),
        out_type=jax.ShapeDtypeStruct((size, 2 * HALF), jnp.int32),
        scratch_types=[
            pltpu.VMEM((CH,), jnp.int32),
            pltpu.VMEM((CH,), jnp.int32),
            pltpu.VMEM((CH, HALF), jnp.int32),
            pltpu.VMEM((CH, HALF), jnp.int32),
            pltpu.SemaphoreType.DMA,
            pltpu.SemaphoreType.DMA,
        ],
    )
    def gk(node_hbm, idx_hbm, out_hbm, idx_v0, idx_v1, rows_v0, rows_v1,
           sem0, sem1):
        w = lax.axis_index("s") * info.num_cores + lax.axis_index("c")

        def issue(cid, idx_base, idx_v, rows_v, sem):
            @pl.when(cid < half)
            def _():
                pltpu.sync_copy(
                    idx_hbm.at[pl.ds(idx_base + cid * CH, CH)], idx_v)
                pltpu.async_copy(node_hbm.at[idx_v], rows_v, sem)

        def drain(cid, lane, idx_v, rows_v, sem):
            @pl.when(cid < half)
            def _():
                pltpu.make_async_copy(
                    node_hbm.at[idx_v], rows_v, sem).wait()
                pltpu.sync_copy(
                    rows_v,
                    out_hbm.at[pl.ds(cid * CH, CH), pl.ds(lane, HALF)])

        issue(w * cpw, src_base, idx_v0, rows_v0, sem0)

        def body(t, carry):
            cid = w * cpw + t
            issue(cid, dst_base, idx_v1, rows_v1, sem1)
            drain(cid, 0, idx_v0, rows_v0, sem0)

            @pl.when(t + 1 < cpw)
            def _():
                issue(cid + 1, src_base, idx_v0, rows_v0, sem0)

            drain(cid, HALF, idx_v1, rows_v1, sem1)
            return carry

        lax.fori_loop(0, cpw, body, 0)

    return gk(node_view, idx_flat)


def _mlp_body(z_ref, g_ref, w1_ref, b1_ref, w2_ref, b2_ref,
              wf_ref, bf_ref, ga_ref, be_ref, prev_ref, o_ref):
    del prev_ref
    x = g_ref[...]
    lo = lax.bitcast_convert_type(x << 16, jnp.float32)
    hi = lax.bitcast_convert_type(x & jnp.int32(-65536), jnp.float32)
    e = jnp.concatenate([z_ref[...], lo, hi], axis=1)
    h = jnp.maximum(
        jnp.dot(e, w1_ref[...], preferred_element_type=jnp.float32)
        + b1_ref[...], 0.0)
    h = jnp.maximum(
        jnp.dot(h, w2_ref[...], preferred_element_type=jnp.float32)
        + b2_ref[...], 0.0)
    o = (jnp.dot(h + e, wf_ref[...], preferred_element_type=jnp.float32)
         + bf_ref[...])
    mu = jnp.mean(o, axis=1, keepdims=True)
    c = o - mu
    var = jnp.mean(c * c, axis=1, keepdims=True)
    o_ref[...] = c * lax.rsqrt(var + 1e-5) * ga_ref[...] + be_ref[...]


def _mlp_body_first(z_ref, g_ref, w1_ref, b1_ref, w2_ref, b2_ref,
                    wf_ref, bf_ref, ga_ref, be_ref, o_ref):
    _mlp_body(z_ref, g_ref, w1_ref, b1_ref, w2_ref, b2_ref,
              wf_ref, bf_ref, ga_ref, be_ref, None, o_ref)


def _mlp_part(z, gathered, weights, prev_out, start, size):
    W1p, b1, W2, b2, Wfp, bf, gamma, beta = weights
    goff = start // TE
    gp = size // TE

    def _const2(shape):
        return pl.BlockSpec(shape, lambda i: (0, 0))

    in_specs = [
        pl.BlockSpec((TE, C_Z), lambda i: (i + goff, 0)),
        pl.BlockSpec((TE, BIAS), lambda i: (i, 0)),
        _const2((HID, HID)),
        _const2((1, HID)),
        _const2((HID, HID)),
        _const2((1, HID)),
        _const2((HID, C_Z)),
        _const2((1, C_Z)),
        _const2((1, C_Z)),
        _const2((1, C_Z)),
    ]
    args = [z, gathered, W1p, b1.reshape(1, HID), W2,
            b2.reshape(1, HID), Wfp, bf.reshape(1, C_Z),
            gamma.reshape(1, C_Z), beta.reshape(1, C_Z)]
    if prev_out is None:
        body = _mlp_body_first
        aliases = {}
    else:
        body = _mlp_body
        in_specs.append(pl.BlockSpec(memory_space=pl.ANY))
        args.append(prev_out)
        aliases = {10: 0}

    return pl.pallas_call(
        body,
        grid=(gp,),
        in_specs=in_specs,
        out_specs=pl.BlockSpec((TE, C_Z), lambda i: (i + goff, 0)),
        out_shape=jax.ShapeDtypeStruct((N_EDGES, C_Z), jnp.float32),
        input_output_aliases=aliases,
        compiler_params=pltpu.CompilerParams(
            dimension_semantics=("parallel",)),
    )(*args)


def kernel(s, z, W0, b0, W1, b1, W2, b2, Wf, bf, gamma, beta, edge_index):
    node_packed = _node_embed(s, W0, b0)
    node_view = node_packed.reshape(N_NODES, HALF)
    idx_flat = _view_idx(edge_index.reshape(-1))
    perm = jnp.asarray(_PERM)
    weights = (W1[perm, :], b1, W2[:, perm], b2[perm], Wf[perm, :], bf,
               gamma, beta)

    starts = [sum(PARTS[:p]) for p in range(len(PARTS))]
    gathered = [_gather_part(node_view, idx_flat, st, sz)
                for st, sz in zip(starts, PARTS)]
    out = None
    for g, st, sz in zip(gathered, starts, PARTS):
        out = _mlp_part(z, g, weights, out, st, sz)
    return out


# speedup vs baseline: 1.1379x; 1.0231x over previous
import functools

import jax
import jax.numpy as jnp
import numpy as np
from jax import lax
from jax.experimental import pallas as pl
from jax.experimental.pallas import tpu as pltpu
from jax.experimental.pallas import tpu_sc as plsc

N_NODES = 10000
N_EDGES = 160000
C_N = 256
C_Z = 128
BIAS = 128
HALF = BIAS // 2
HID = 384

PARTS = (16000, 32000, 48000, 64000)
TE = 2000
CH = 128

_PERM = np.concatenate([
    np.arange(0, 128),
    np.arange(128, 192),
    np.arange(256, 320),
    np.arange(192, 256),
    np.arange(320, 384),
])


def _rtne16(u):
    return (u + 0x7FFF + ((u >> 16) & 1)) >> 16


def _embed_body(s_ref, w0_ref, b0_ref, o_ref):
    o = (jnp.dot(s_ref[...], w0_ref[...], preferred_element_type=jnp.float32)
         + b0_ref[...])
    u = lax.bitcast_convert_type(o, jnp.int32)
    packed = ((_rtne16(u[:, :HALF]) & 0xFFFF)
              | (_rtne16(u[:, HALF:]) << 16))
    tn_half = packed.shape[0] // 2
    o_ref[...] = jnp.concatenate(
        [packed[:tn_half, :], packed[tn_half:, :]], axis=1)


def _node_embed(s, W0, b0):
    TN = 2000
    return pl.pallas_call(
        _embed_body,
        grid=(N_NODES // TN,),
        in_specs=[
            pl.BlockSpec((TN, C_N), lambda i: (i, 0)),
            pl.BlockSpec((C_N, BIAS), lambda i: (0, 0)),
            pl.BlockSpec((1, BIAS), lambda i: (0, 0)),
        ],
        out_specs=pl.BlockSpec((TN // 2, BIAS), lambda i: (i, 0)),
        out_shape=jax.ShapeDtypeStruct((N_NODES // 2, BIAS), jnp.int32),
    )(s, W0, b0.reshape(1, BIAS))


def _view_idx(idx):
    b = idx // 2000
    r = idx % 2000
    return b * 2000 + jnp.where(r < 1000, 2 * r, 2 * (r - 1000) + 1)


def _gather_part(node_view, idx_flat, start, size):
    info = plsc.get_sparse_core_info()
    nw = info.num_cores * info.num_subcores
    half = size // CH
    cpw = (half + nw - 1) // nw
    src_base = start
    dst_base = N_EDGES + start
    mesh = plsc.VectorSubcoreMesh(core_axis_name="c", subcore_axis_name="s")

    @functools.partial(
        pl.kernel,
        mesh=mesh,
        compiler_params=pltpu.CompilerParams(use_tc_tiling_on_sc=False),
        out_type=jax.ShapeDtypeStruct((size, 2 * HALF), jnp.int32),
        scratch_types=[
            pltpu.VMEM((CH,), jnp.int32),
            pltpu.VMEM((CH,), jnp.int32),
            pltpu.VMEM((CH, HALF), jnp.int32),
            pltpu.VMEM((CH, HALF), jnp.int32),
            pltpu.SemaphoreType.DMA,
            pltpu.SemaphoreType.DMA,
        ],
    )
    def gk(node_hbm, idx_hbm, out_hbm, idx_v0, idx_v1, rows_v0, rows_v1,
           sem0, sem1):
        w = lax.axis_index("s") * info.num_cores + lax.axis_index("c")

        def issue(cid, idx_base, idx_v, rows_v, sem):
            @pl.when(cid < half)
            def _():
                pltpu.sync_copy(
                    idx_hbm.at[pl.ds(idx_base + cid * CH, CH)], idx_v)
                for j in range(CH // 16):
                    n = idx_v[pl.ds(j * 16, 16)]
                    b = (n * 33555) >> 26
                    r = n - b * 2000
                    idx_v[pl.ds(j * 16, 16)] = (
                        b * 2000 + 2 * r - jnp.where(r < 1000, 0, 1999))
                pltpu.async_copy(node_hbm.at[idx_v], rows_v, sem)

        def drain(cid, lane, idx_v, rows_v, sem):
            @pl.when(cid < half)
            def _():
                pltpu.make_async_copy(
                    node_hbm.at[idx_v], rows_v, sem).wait()
                pltpu.sync_copy(
                    rows_v,
                    out_hbm.at[pl.ds(cid * CH, CH), pl.ds(lane, HALF)])

        issue(w * cpw, src_base, idx_v0, rows_v0, sem0)

        def body(t, carry):
            cid = w * cpw + t
            issue(cid, dst_base, idx_v1, rows_v1, sem1)
            drain(cid, 0, idx_v0, rows_v0, sem0)

            @pl.when(t + 1 < cpw)
            def _():
                issue(cid + 1, src_base, idx_v0, rows_v0, sem0)

            drain(cid, HALF, idx_v1, rows_v1, sem1)
            return carry

        lax.fori_loop(0, cpw, body, 0)

    return gk(node_view, idx_flat)


def _mlp_body(z_ref, g_ref, w1_ref, b1_ref, w2_ref, b2_ref,
              wf_ref, bf_ref, ga_ref, be_ref, prev_ref, o_ref):
    del prev_ref
    x = g_ref[...]
    lo = lax.bitcast_convert_type(x << 16, jnp.float32)
    hi = lax.bitcast_convert_type(x & jnp.int32(-65536), jnp.float32)
    e = jnp.concatenate([z_ref[...], lo, hi], axis=1)
    h = jnp.maximum(
        jnp.dot(e, w1_ref[...], preferred_element_type=jnp.float32)
        + b1_ref[...], 0.0)
    h = jnp.maximum(
        jnp.dot(h, w2_ref[...], preferred_element_type=jnp.float32)
        + b2_ref[...], 0.0)
    o = (jnp.dot(h + e, wf_ref[...], preferred_element_type=jnp.float32)
         + bf_ref[...])
    mu = jnp.mean(o, axis=1, keepdims=True)
    c = o - mu
    var = jnp.mean(c * c, axis=1, keepdims=True)
    o_ref[...] = c * lax.rsqrt(var + 1e-5) * ga_ref[...] + be_ref[...]


def _mlp_body_first(z_ref, g_ref, w1_ref, b1_ref, w2_ref, b2_ref,
                    wf_ref, bf_ref, ga_ref, be_ref, o_ref):
    _mlp_body(z_ref, g_ref, w1_ref, b1_ref, w2_ref, b2_ref,
              wf_ref, bf_ref, ga_ref, be_ref, None, o_ref)


def _mlp_part(z, gathered, weights, prev_out, start, size):
    W1p, b1, W2, b2, Wfp, bf, gamma, beta = weights
    goff = start // TE
    gp = size // TE

    def _const2(shape):
        return pl.BlockSpec(shape, lambda i: (0, 0))

    in_specs = [
        pl.BlockSpec((TE, C_Z), lambda i: (i + goff, 0)),
        pl.BlockSpec((TE, BIAS), lambda i: (i, 0)),
        _const2((HID, HID)),
        _const2((1, HID)),
        _const2((HID, HID)),
        _const2((1, HID)),
        _const2((HID, C_Z)),
        _const2((1, C_Z)),
        _const2((1, C_Z)),
        _const2((1, C_Z)),
    ]
    args = [z, gathered, W1p, b1.reshape(1, HID), W2,
            b2.reshape(1, HID), Wfp, bf.reshape(1, C_Z),
            gamma.reshape(1, C_Z), beta.reshape(1, C_Z)]
    if prev_out is None:
        body = _mlp_body_first
        aliases = {}
    else:
        body = _mlp_body
        in_specs.append(pl.BlockSpec(memory_space=pl.ANY))
        args.append(prev_out)
        aliases = {10: 0}

    return pl.pallas_call(
        body,
        grid=(gp,),
        in_specs=in_specs,
        out_specs=pl.BlockSpec((TE, C_Z), lambda i: (i + goff, 0)),
        out_shape=jax.ShapeDtypeStruct((N_EDGES, C_Z), jnp.float32),
        input_output_aliases=aliases,
        compiler_params=pltpu.CompilerParams(
            dimension_semantics=("parallel",)),
    )(*args)


def kernel(s, z, W0, b0, W1, b1, W2, b2, Wf, bf, gamma, beta, edge_index):
    node_packed = _node_embed(s, W0, b0)
    node_view = node_packed.reshape(N_NODES, HALF)
    idx_flat = edge_index.reshape(-1)
    perm = jnp.asarray(_PERM)
    weights = (W1[perm, :], b1, W2[:, perm], b2[perm], Wf[perm, :], bf,
               gamma, beta)

    starts = [sum(PARTS[:p]) for p in range(len(PARTS))]
    gathered = [_gather_part(node_view, idx_flat, st, sz)
                for st, sz in zip(starts, PARTS)]
    out = None
    for g, st, sz in zip(gathered, starts, PARTS):
        out = _mlp_part(z, g, weights, out, st, sz)
    return out
